# baseline (speedup 1.0000x reference)
import jax
import jax.numpy as jnp
from jax import lax
from jax.experimental import pallas as pl
from jax.experimental.pallas import tpu as pltpu
from jax.experimental.pallas import tpu_sc as plsc

N = 10000
E = 320000
D = 128
NC = 2
NS = 16
NW = NC * NS
CHUNK = 128
NBUF = 2
NIB = 4
EPW = 10240
NCHUNKS = EPW // CHUNK
NCH_DATA = NCHUNKS + NIB
N_ACC = 10112
RPT = N_ACC // NS
CW = 16


def _make_edge_pass(with_cnt: bool):
  mesh = plsc.VectorSubcoreMesh(core_axis_name="c", subcore_axis_name="s")
  out_type = [jax.ShapeDtypeStruct((NC, N_ACC, D), jnp.float32)]
  if with_cnt:
    out_type.append(jax.ShapeDtypeStruct((NC * N_ACC,), jnp.float32))
  scratch = (
      [
          pltpu.VMEM((NIB, CHUNK), jnp.int32),
          pltpu.VMEM((NIB, CHUNK), jnp.int32),
          pltpu.VMEM((NBUF, CHUNK, D), jnp.float32),
          pltpu.VMEM((CHUNK,), jnp.float32),
          pltpu.VMEM((RPT,), jnp.float32),
          pltpu.VMEM_SHARED((N_ACC, D), jnp.float32),
          pltpu.VMEM_SHARED((N_ACC,), jnp.float32),
      ]
      + [pltpu.SemaphoreType.DMA] * (NBUF + 2 * NIB)
  )

  def body(p_hbm, src_hbm, dst_hbm, zrow_hbm, zcnt_hbm, ones_hbm, *refs):
    if with_cnt:
      out_hbm, cnt_hbm = refs[0], refs[1]
      refs = refs[2:]
    else:
      out_hbm = refs[0]
      cnt_hbm = None
      refs = refs[1:]
    idx_s, idx_d, rows, ones_v, cbuf, s_sh, c_sh = refs[:7]
    sems = refs[7:]
    gsem = sems[:NBUF]
    isem_s = sems[NBUF:NBUF + NIB]
    isem_d = sems[NBUF + NIB:]

    cid = lax.axis_index("c")
    sid = lax.axis_index("s")
    wid = sid * NC + cid
    row0 = sid * RPT
    base = wid * NCH_DATA * CHUNK

    def idx_load(j, q):
      off = base + j * CHUNK
      pltpu.async_copy(src_hbm.at[pl.ds(off, CHUNK)], idx_s.at[q], isem_s[q])
      pltpu.async_copy(dst_hbm.at[pl.ds(off, CHUNK)], idx_d.at[q], isem_d[q])

    def idx_wait(j, q):
      off = base + j * CHUNK
      pltpu.make_async_copy(src_hbm.at[pl.ds(off, CHUNK)], idx_s.at[q],
                            isem_s[q]).wait()
      pltpu.make_async_copy(dst_hbm.at[pl.ds(off, CHUNK)], idx_d.at[q],
                            isem_d[q]).wait()

    def gather(q, b):
      pltpu.async_copy(p_hbm.at[idx_s.at[q]], rows.at[b], gsem[b])

    def gather_wait(q, b):
      pltpu.make_async_copy(p_hbm.at[idx_s.at[q]], rows.at[b],
                            gsem[b]).wait()

    for q in range(NIB):
      idx_load(q, q)
    pltpu.sync_copy(zrow_hbm.at[pl.ds(row0, RPT)], s_sh.at[pl.ds(row0, RPT)])
    if with_cnt:
      pltpu.sync_copy(zcnt_hbm.at[pl.ds(row0, RPT)], cbuf)
      pltpu.sync_copy(cbuf, c_sh.at[pl.ds(row0, RPT)])
      pltpu.sync_copy(ones_hbm, ones_v)
    plsc.subcore_barrier()

    for b in range(NBUF):
      idx_wait(b, b)
      gather(b, b)

    @pl.loop(0, NCHUNKS, step=NIB)
    def _steps(g0):
      for q in range(NIB):
        j = g0 + q
        b = q % NBUF
        q2 = (q + NBUF) % NIB
        gather_wait(q, b)
        pltpu.sync_copy(rows.at[b], s_sh.at[idx_d.at[q]], add=True)
        if with_cnt:
          pltpu.sync_copy(ones_v, c_sh.at[idx_d.at[q]], add=True)
        idx_load(j + NIB, q)
        idx_wait(j + NBUF, q2)
        gather(q2, b)

    for b in range(NBUF):
      gather_wait((NCHUNKS + b) % NIB, b)
    for q in range(NBUF, NIB):
      idx_wait(NCHUNKS + q, q)

    plsc.subcore_barrier()
    pltpu.sync_copy(s_sh.at[pl.ds(row0, RPT)],
                    out_hbm.at[cid, pl.ds(row0, RPT)])
    if with_cnt:
      pltpu.sync_copy(c_sh.at[pl.ds(row0, RPT)], cbuf)
      pltpu.sync_copy(cbuf, cnt_hbm.at[pl.ds(cid * N_ACC + row0, RPT)])

  return pl.kernel(body, out_type=out_type, mesh=mesh, scratch_types=scratch)


_edge_pass_cnt = _make_edge_pass(True)
_edge_pass = _make_edge_pass(False)


BN = 2000


def _pre_body(x_ref, wl_ref, wr_ref, b_ref, t_ref, r_ref):
  x = x_ref[...]
  t_ref[...] = jnp.dot(x, wl_ref[...], preferred_element_type=jnp.float32)
  r_ref[...] = jnp.dot(x, wr_ref[...],
                       preferred_element_type=jnp.float32) + b_ref[...]


def _mid_body(s0_ref, s1_ref, c0_ref, c1_ref, r_ref, wl_ref, wr_ref, b_ref,
              t_ref, ro_ref):
  cnt = jnp.maximum(c0_ref[:, 0:1] + c1_ref[:, 0:1], 1.0)
  h = jnp.maximum((s0_ref[...] + s1_ref[...]) / cnt + r_ref[...], 0.0)
  t_ref[...] = jnp.dot(h, wl_ref[...], preferred_element_type=jnp.float32)
  ro_ref[...] = jnp.dot(h, wr_ref[...],
                        preferred_element_type=jnp.float32) + b_ref[...]


def _head_body(s0_ref, s1_ref, c0_ref, c1_ref, r_ref, wh1_ref, bh1_ref,
               wh2_ref, bh2_ref, out_ref):
  cnt = jnp.maximum(c0_ref[:, 0:1] + c1_ref[:, 0:1], 1.0)
  h = jnp.maximum((s0_ref[...] + s1_ref[...]) / cnt + r_ref[...], 0.0)
  h = jnp.maximum(jnp.dot(h, wh1_ref[...],
                          preferred_element_type=jnp.float32) + bh1_ref[...],
                  0.0)
  out_ref[...] = jnp.dot(h, wh2_ref[...],
                         preferred_element_type=jnp.float32) + bh2_ref[...]


def _row_spec(w):
  return pl.BlockSpec((BN, w), lambda i: (i, 0))


def _full_spec(shape):
  return pl.BlockSpec(shape, lambda i: (0,) * len(shape))


_GRID = N // BN

_pre = pl.pallas_call(
    _pre_body,
    grid=(_GRID,),
    in_specs=[_row_spec(D), _full_spec((D, D)), _full_spec((D, D)),
              _full_spec((1, D))],
    out_specs=[_row_spec(D), _row_spec(D)],
    out_shape=[jax.ShapeDtypeStruct((N, D), jnp.float32)] * 2,
)

_mid = pl.pallas_call(
    _mid_body,
    grid=(_GRID,),
    in_specs=[_row_spec(D), _row_spec(D), _row_spec(1), _row_spec(1),
              _row_spec(D), _full_spec((D, D)), _full_spec((D, D)),
              _full_spec((1, D))],
    out_specs=[_row_spec(D), _row_spec(D)],
    out_shape=[jax.ShapeDtypeStruct((N, D), jnp.float32)] * 2,
)

_head = pl.pallas_call(
    _head_body,
    grid=(_GRID,),
    in_specs=[_row_spec(D), _row_spec(D), _row_spec(1), _row_spec(1),
              _row_spec(D), _full_spec((D, D // 2)), _full_spec((1, D // 2)),
              _full_spec((D // 2, 4)), _full_spec((1, 4))],
    out_specs=_row_spec(4),
    out_shape=jax.ShapeDtypeStruct((N, 4), jnp.float32),
)


def kernel(x, edge_index, W_l0, b_l0, W_r0, W_l1, b_l1, W_r1, W_l2, b_l2,
           W_r2, Wh1, bh1, Wh2, bh2):
  src = edge_index[0].astype(jnp.int32)
  dst = edge_index[1].astype(jnp.int32)
  pad = NW * EPW - E
  src_p = jnp.concatenate([src, jnp.zeros((pad,), jnp.int32)]).reshape(NW, EPW)
  dst_p = jnp.concatenate([dst, jnp.full((pad,), N, jnp.int32)]).reshape(NW, EPW)
  ppad = NIB * CHUNK
  src_p = jnp.concatenate(
      [src_p, jnp.zeros((NW, ppad), jnp.int32)], axis=1).reshape(-1)
  dst_p = jnp.concatenate(
      [dst_p, jnp.full((NW, ppad), N, jnp.int32)], axis=1).reshape(-1)
  zrow = jnp.zeros((N_ACC, D), jnp.float32)
  zcnt = jnp.zeros((N_ACC,), jnp.float32)
  ones = jnp.ones((CHUNK,), jnp.float32)

  t0, r0 = _pre(x, W_l0, W_r0, b_l0.reshape(1, D))
  s, c = _edge_pass_cnt(t0, src_p, dst_p, zrow, zcnt, ones)
  c = c.reshape(NC, N_ACC)
  c0, c1 = c[0, :N].reshape(N, 1), c[1, :N].reshape(N, 1)

  t1, r1 = _mid(s[0, :N], s[1, :N], c0, c1, r0, W_l1, W_r1,
                b_l1.reshape(1, D))
  (s,) = _edge_pass(t1, src_p, dst_p, zrow, zcnt, ones)
  t2, r2 = _mid(s[0, :N], s[1, :N], c0, c1, r1, W_l2, W_r2,
                b_l2.reshape(1, D))
  (s,) = _edge_pass(t2, src_p, dst_p, zrow, zcnt, ones)
  out = _head(s[0, :N], s[1, :N], c0, c1, r2, Wh1, bh1.reshape(1, D // 2),
              Wh2, bh2.reshape(1, 4))
  return out


# speedup vs baseline: 1.8805x; 1.8805x over previous
import jax
import jax.numpy as jnp
from jax import lax
from jax.experimental import pallas as pl
from jax.experimental.pallas import tpu as pltpu
from jax.experimental.pallas import tpu_sc as plsc

N = 10000
E = 320000
D = 128
NC = 2
NS = 16
NW = NC * NS
CHUNK = 128
NBUF = 2
EPW = 10240
NCHUNKS = EPW // CHUNK
N_ACC = 10112
RPT = N_ACC // NS
CW = 16


def _make_edge_pass(with_cnt: bool):
  mesh = plsc.VectorSubcoreMesh(core_axis_name="c", subcore_axis_name="s")
  out_type = [jax.ShapeDtypeStruct((NC, N_ACC, D), jnp.float32)]
  if with_cnt:
    out_type.append(jax.ShapeDtypeStruct((NC * N_ACC,), jnp.float32))
  scratch = [
      pltpu.VMEM((NBUF, CHUNK), jnp.int32),
      pltpu.VMEM((NBUF, CHUNK), jnp.int32),
      pltpu.VMEM((NBUF, CHUNK, D), jnp.float32),
      pltpu.VMEM((CHUNK,), jnp.float32),
      pltpu.VMEM((RPT,), jnp.float32),
      pltpu.VMEM_SHARED((N_ACC, D), jnp.float32),
      pltpu.VMEM_SHARED((N_ACC,), jnp.float32),
  ] + [pltpu.SemaphoreType.DMA] * NBUF

  def body(p_hbm, src_hbm, dst_hbm, zrow_hbm, zcnt_hbm, ones_hbm, *refs):
    if with_cnt:
      out_hbm, cnt_hbm = refs[0], refs[1]
      refs = refs[2:]
    else:
      out_hbm = refs[0]
      cnt_hbm = None
      refs = refs[1:]
    idx_s, idx_d, rows, ones_v, cbuf, s_sh, c_sh = refs[:7]
    gsem = refs[7:]

    cid = lax.axis_index("c")
    sid = lax.axis_index("s")
    wid = sid * NC + cid
    row0 = sid * RPT
    base = wid * EPW

    pltpu.sync_copy(zrow_hbm.at[pl.ds(row0, RPT)], s_sh.at[pl.ds(row0, RPT)])
    if with_cnt:
      pltpu.sync_copy(zcnt_hbm.at[pl.ds(row0, RPT)], cbuf)
      pltpu.sync_copy(cbuf, c_sh.at[pl.ds(row0, RPT)])
      pltpu.sync_copy(ones_hbm, ones_v)
    plsc.subcore_barrier()

    def start(g, b):
      off = base + g * CHUNK
      pltpu.sync_copy(src_hbm.at[pl.ds(off, CHUNK)], idx_s.at[b])
      pltpu.sync_copy(dst_hbm.at[pl.ds(off, CHUNK)], idx_d.at[b])
      pltpu.async_copy(p_hbm.at[idx_s.at[b]], rows.at[b], gsem[b])

    for b in range(NBUF):
      start(b, b)

    @pl.loop(0, NCHUNKS, step=NBUF)
    def _steps(g0):
      for b in range(NBUF):
        g = g0 + b
        pltpu.make_async_copy(p_hbm.at[idx_s.at[b]], rows.at[b],
                              gsem[b]).wait()
        pltpu.sync_copy(rows.at[b], s_sh.at[idx_d.at[b]], add=True)
        if with_cnt:
          pltpu.sync_copy(ones_v, c_sh.at[idx_d.at[b]], add=True)

        @pl.when(g + NBUF < NCHUNKS)
        def _():
          start(g + NBUF, b)

    plsc.subcore_barrier()
    pltpu.sync_copy(s_sh.at[pl.ds(row0, RPT)],
                    out_hbm.at[cid, pl.ds(row0, RPT)])
    if with_cnt:
      pltpu.sync_copy(c_sh.at[pl.ds(row0, RPT)], cbuf)
      pltpu.sync_copy(cbuf, cnt_hbm.at[pl.ds(cid * N_ACC + row0, RPT)])

  return pl.kernel(body, out_type=out_type, mesh=mesh, scratch_types=scratch)


_edge_pass_cnt = _make_edge_pass(True)
_edge_pass = _make_edge_pass(False)


BN = 2000


def _pre_body(x_ref, wl_ref, wr_ref, b_ref, t_ref, r_ref):
  x = x_ref[...]
  t_ref[...] = jnp.dot(x, wl_ref[...], preferred_element_type=jnp.float32)
  r_ref[...] = jnp.dot(x, wr_ref[...],
                       preferred_element_type=jnp.float32) + b_ref[...]


def _mid_body(s0_ref, s1_ref, c0_ref, c1_ref, r_ref, wl_ref, wr_ref, b_ref,
              t_ref, ro_ref):
  cnt = jnp.maximum(c0_ref[:, 0:1] + c1_ref[:, 0:1], 1.0)
  h = jnp.maximum((s0_ref[...] + s1_ref[...]) / cnt + r_ref[...], 0.0)
  t_ref[...] = jnp.dot(h, wl_ref[...], preferred_element_type=jnp.float32)
  ro_ref[...] = jnp.dot(h, wr_ref[...],
                        preferred_element_type=jnp.float32) + b_ref[...]


def _head_body(s0_ref, s1_ref, c0_ref, c1_ref, r_ref, wh1_ref, bh1_ref,
               wh2_ref, bh2_ref, out_ref):
  cnt = jnp.maximum(c0_ref[:, 0:1] + c1_ref[:, 0:1], 1.0)
  h = jnp.maximum((s0_ref[...] + s1_ref[...]) / cnt + r_ref[...], 0.0)
  h = jnp.maximum(jnp.dot(h, wh1_ref[...],
                          preferred_element_type=jnp.float32) + bh1_ref[...],
                  0.0)
  out_ref[...] = jnp.dot(h, wh2_ref[...],
                         preferred_element_type=jnp.float32) + bh2_ref[...]


def _row_spec(w):
  return pl.BlockSpec((BN, w), lambda i: (i, 0))


def _full_spec(shape):
  return pl.BlockSpec(shape, lambda i: (0,) * len(shape))


_GRID = N // BN

_pre = pl.pallas_call(
    _pre_body,
    grid=(_GRID,),
    in_specs=[_row_spec(D), _full_spec((D, D)), _full_spec((D, D)),
              _full_spec((1, D))],
    out_specs=[_row_spec(D), _row_spec(D)],
    out_shape=[jax.ShapeDtypeStruct((N, D), jnp.float32)] * 2,
)

_mid = pl.pallas_call(
    _mid_body,
    grid=(_GRID,),
    in_specs=[_row_spec(D), _row_spec(D), _row_spec(1), _row_spec(1),
              _row_spec(D), _full_spec((D, D)), _full_spec((D, D)),
              _full_spec((1, D))],
    out_specs=[_row_spec(D), _row_spec(D)],
    out_shape=[jax.ShapeDtypeStruct((N, D), jnp.float32)] * 2,
)

_head = pl.pallas_call(
    _head_body,
    grid=(_GRID,),
    in_specs=[_row_spec(D), _row_spec(D), _row_spec(1), _row_spec(1),
              _row_spec(D), _full_spec((D, D // 2)), _full_spec((1, D // 2)),
              _full_spec((D // 2, 4)), _full_spec((1, 4))],
    out_specs=_row_spec(4),
    out_shape=jax.ShapeDtypeStruct((N, 4), jnp.float32),
)


def kernel(x, edge_index, W_l0, b_l0, W_r0, W_l1, b_l1, W_r1, W_l2, b_l2,
           W_r2, Wh1, bh1, Wh2, bh2):
  src = edge_index[0].astype(jnp.int32)
  dst = edge_index[1].astype(jnp.int32)
  pad = NW * EPW - E
  src_p = jnp.concatenate([src, jnp.zeros((pad,), jnp.int32)])
  dst_p = jnp.concatenate([dst, jnp.full((pad,), N, jnp.int32)])
  zrow = jnp.zeros((N_ACC, D), jnp.float32)
  zcnt = jnp.zeros((N_ACC,), jnp.float32)
  ones = jnp.ones((CHUNK,), jnp.float32)

  t0, r0 = _pre(x, W_l0, W_r0, b_l0.reshape(1, D))
  s, c = _edge_pass_cnt(t0, src_p, dst_p, zrow, zcnt, ones)
  c = c.reshape(NC, N_ACC)
  c0, c1 = c[0, :N].reshape(N, 1), c[1, :N].reshape(N, 1)

  t1, r1 = _mid(s[0, :N], s[1, :N], c0, c1, r0, W_l1, W_r1,
                b_l1.reshape(1, D))
  (s,) = _edge_pass(t1, src_p, dst_p, zrow, zcnt, ones)
  t2, r2 = _mid(s[0, :N], s[1, :N], c0, c1, r1, W_l2, W_r2,
                b_l2.reshape(1, D))
  (s,) = _edge_pass(t2, src_p, dst_p, zrow, zcnt, ones)
  out = _head(s[0, :N], s[1, :N], c0, c1, r2, Wh1, bh1.reshape(1, D // 2),
              Wh2, bh2.reshape(1, 4))
  return out


# speedup vs baseline: 2.0324x; 1.0808x over previous
import jax
import jax.numpy as jnp
from jax import lax
from jax.experimental import pallas as pl
from jax.experimental.pallas import tpu as pltpu
from jax.experimental.pallas import tpu_sc as plsc

N = 10000
E = 320000
D = 128
NC = 2
NS = 16
NW = NC * NS
CHUNK = 128
NBUF = 2
EPW = 10240
NCHUNKS = EPW // CHUNK
NCH0 = 104
NCH1 = 2 * NCHUNKS - NCH0
N_ACC = 10112
RPT = N_ACC // NS
CW = 16


def _make_edge_pass(with_cnt: bool):
  mesh = plsc.VectorSubcoreMesh(core_axis_name="c", subcore_axis_name="s")
  out_type = [jax.ShapeDtypeStruct((NC, N_ACC, D), jnp.float32)]
  if with_cnt:
    out_type.append(jax.ShapeDtypeStruct((NC * N_ACC,), jnp.float32))
  scratch = [
      pltpu.VMEM((NBUF, CHUNK), jnp.int32),
      pltpu.VMEM((NBUF, CHUNK), jnp.int32),
      pltpu.VMEM((NBUF, CHUNK, D), jnp.float32),
      pltpu.VMEM((CHUNK,), jnp.float32),
      pltpu.VMEM((RPT,), jnp.float32),
      pltpu.VMEM_SHARED((N_ACC, D), jnp.float32),
      pltpu.VMEM_SHARED((N_ACC,), jnp.float32),
  ] + [pltpu.SemaphoreType.DMA] * NBUF

  def body(p_hbm, src_hbm, dst_hbm, zrow_hbm, zcnt_hbm, ones_hbm, *refs):
    if with_cnt:
      out_hbm, cnt_hbm = refs[0], refs[1]
      refs = refs[2:]
    else:
      out_hbm = refs[0]
      cnt_hbm = None
      refs = refs[1:]
    idx_s, idx_d, rows, ones_v, cbuf, s_sh, c_sh = refs[:7]
    gsem = refs[7:]

    cid = lax.axis_index("c")
    sid = lax.axis_index("s")
    row0 = sid * RPT
    base = (sid * 2 * NCHUNKS + cid * NCH0) * CHUNK
    nch = jnp.where(cid == 0, NCH0, NCH1)

    pltpu.sync_copy(zrow_hbm.at[pl.ds(row0, RPT)], s_sh.at[pl.ds(row0, RPT)])
    if with_cnt:
      pltpu.sync_copy(zcnt_hbm.at[pl.ds(row0, RPT)], cbuf)
      pltpu.sync_copy(cbuf, c_sh.at[pl.ds(row0, RPT)])
      pltpu.sync_copy(ones_hbm, ones_v)
    plsc.subcore_barrier()

    def start(g, b):
      off = base + g * CHUNK
      pltpu.sync_copy(src_hbm.at[pl.ds(off, CHUNK)], idx_s.at[b])
      pltpu.sync_copy(dst_hbm.at[pl.ds(off, CHUNK)], idx_d.at[b])
      pltpu.async_copy(p_hbm.at[idx_s.at[b]], rows.at[b], gsem[b])

    for b in range(NBUF):
      start(b, b)

    @pl.loop(0, nch, step=NBUF)
    def _steps(g0):
      for b in range(NBUF):
        g = g0 + b
        pltpu.make_async_copy(p_hbm.at[idx_s.at[b]], rows.at[b],
                              gsem[b]).wait()
        pltpu.sync_copy(rows.at[b], s_sh.at[idx_d.at[b]], add=True)
        if with_cnt:
          pltpu.sync_copy(ones_v, c_sh.at[idx_d.at[b]], add=True)

        @pl.when(g + NBUF < nch)
        def _():
          start(g + NBUF, b)

    plsc.subcore_barrier()
    pltpu.sync_copy(s_sh.at[pl.ds(row0, RPT)],
                    out_hbm.at[cid, pl.ds(row0, RPT)])
    if with_cnt:
      pltpu.sync_copy(c_sh.at[pl.ds(row0, RPT)], cbuf)
      pltpu.sync_copy(cbuf, cnt_hbm.at[pl.ds(cid * N_ACC + row0, RPT)])

  return pl.kernel(body, out_type=out_type, mesh=mesh, scratch_types=scratch)


_edge_pass_cnt = _make_edge_pass(True)
_edge_pass = _make_edge_pass(False)


BN = 2000


def _pre_body(x_ref, wl_ref, wr_ref, b_ref, t_ref, r_ref):
  x = x_ref[...]
  t_ref[...] = jnp.dot(x, wl_ref[...], preferred_element_type=jnp.float32)
  r_ref[...] = jnp.dot(x, wr_ref[...],
                       preferred_element_type=jnp.float32) + b_ref[...]


def _mid_body(s0_ref, s1_ref, c0_ref, c1_ref, r_ref, wl_ref, wr_ref, b_ref,
              t_ref, ro_ref):
  cnt = jnp.maximum(c0_ref[:, 0:1] + c1_ref[:, 0:1], 1.0)
  h = jnp.maximum((s0_ref[...] + s1_ref[...]) / cnt + r_ref[...], 0.0)
  t_ref[...] = jnp.dot(h, wl_ref[...], preferred_element_type=jnp.float32)
  ro_ref[...] = jnp.dot(h, wr_ref[...],
                        preferred_element_type=jnp.float32) + b_ref[...]


def _head_body(s0_ref, s1_ref, c0_ref, c1_ref, r_ref, wh1_ref, bh1_ref,
               wh2_ref, bh2_ref, out_ref):
  cnt = jnp.maximum(c0_ref[:, 0:1] + c1_ref[:, 0:1], 1.0)
  h = jnp.maximum((s0_ref[...] + s1_ref[...]) / cnt + r_ref[...], 0.0)
  h = jnp.maximum(jnp.dot(h, wh1_ref[...],
                          preferred_element_type=jnp.float32) + bh1_ref[...],
                  0.0)
  out_ref[...] = jnp.dot(h, wh2_ref[...],
                         preferred_element_type=jnp.float32) + bh2_ref[...]


def _row_spec(w):
  return pl.BlockSpec((BN, w), lambda i: (i, 0))


def _full_spec(shape):
  return pl.BlockSpec(shape, lambda i: (0,) * len(shape))


_GRID = N // BN

_pre = pl.pallas_call(
    _pre_body,
    grid=(_GRID,),
    in_specs=[_row_spec(D), _full_spec((D, D)), _full_spec((D, D)),
              _full_spec((1, D))],
    out_specs=[_row_spec(D), _row_spec(D)],
    out_shape=[jax.ShapeDtypeStruct((N, D), jnp.float32)] * 2,
)

_mid = pl.pallas_call(
    _mid_body,
    grid=(_GRID,),
    in_specs=[_row_spec(D), _row_spec(D), _row_spec(1), _row_spec(1),
              _row_spec(D), _full_spec((D, D)), _full_spec((D, D)),
              _full_spec((1, D))],
    out_specs=[_row_spec(D), _row_spec(D)],
    out_shape=[jax.ShapeDtypeStruct((N, D), jnp.float32)] * 2,
)

_head = pl.pallas_call(
    _head_body,
    grid=(_GRID,),
    in_specs=[_row_spec(D), _row_spec(D), _row_spec(1), _row_spec(1),
              _row_spec(D), _full_spec((D, D // 2)), _full_spec((1, D // 2)),
              _full_spec((D // 2, 4)), _full_spec((1, 4))],
    out_specs=_row_spec(4),
    out_shape=jax.ShapeDtypeStruct((N, 4), jnp.float32),
)


def kernel(x, edge_index, W_l0, b_l0, W_r0, W_l1, b_l1, W_r1, W_l2, b_l2,
           W_r2, Wh1, bh1, Wh2, bh2):
  src = edge_index[0].astype(jnp.int32)
  dst = edge_index[1].astype(jnp.int32)
  pad = NW * EPW - E
  src_p = jnp.concatenate([src, jnp.zeros((pad,), jnp.int32)])
  dst_p = jnp.concatenate([dst, jnp.full((pad,), N, jnp.int32)])
  zrow = jnp.zeros((N_ACC, D), jnp.float32)
  zcnt = jnp.zeros((N_ACC,), jnp.float32)
  ones = jnp.ones((CHUNK,), jnp.float32)

  t0, r0 = _pre(x, W_l0, W_r0, b_l0.reshape(1, D))
  s, c = _edge_pass_cnt(t0, src_p, dst_p, zrow, zcnt, ones)
  c = c.reshape(NC, N_ACC)
  c0, c1 = c[0, :N].reshape(N, 1), c[1, :N].reshape(N, 1)

  t1, r1 = _mid(s[0, :N], s[1, :N], c0, c1, r0, W_l1, W_r1,
                b_l1.reshape(1, D))
  (s,) = _edge_pass(t1, src_p, dst_p, zrow, zcnt, ones)
  t2, r2 = _mid(s[0, :N], s[1, :N], c0, c1, r1, W_l2, W_r2,
                b_l2.reshape(1, D))
  (s,) = _edge_pass(t2, src_p, dst_p, zrow, zcnt, ones)
  out = _head(s[0, :N], s[1, :N], c0, c1, r2, Wh1, bh1.reshape(1, D // 2),
              Wh2, bh2.reshape(1, 4))
  return out


# speedup vs baseline: 2.0440x; 1.0057x over previous
import jax
import jax.numpy as jnp
from jax import lax
from jax.experimental import pallas as pl
from jax.experimental.pallas import tpu as pltpu
from jax.experimental.pallas import tpu_sc as plsc

N = 10000
E = 320000
D = 128
NC = 2
NS = 16
NW = NC * NS
CHUNK = 128
NBUF = 2
EPW = 10240
NCHUNKS = EPW // CHUNK
NCH0 = 124
NCH1 = 2 * NCHUNKS - NCH0
N_ACC = 10112
RPT = N_ACC // NS
CW = 16


def _make_edge_pass(with_cnt: bool):
  mesh = plsc.VectorSubcoreMesh(core_axis_name="c", subcore_axis_name="s")
  out_type = [jax.ShapeDtypeStruct((NC, N_ACC, D), jnp.float32)]
  if with_cnt:
    out_type.append(jax.ShapeDtypeStruct((NC * N_ACC,), jnp.float32))
  scratch = [
      pltpu.VMEM((NBUF, CHUNK), jnp.int32),
      pltpu.VMEM((NBUF, CHUNK), jnp.int32),
      pltpu.VMEM((NBUF, CHUNK, D), jnp.float32),
      pltpu.VMEM((CHUNK,), jnp.float32),
      pltpu.VMEM((RPT,), jnp.float32),
      pltpu.VMEM_SHARED((N_ACC, D), jnp.float32),
      pltpu.VMEM_SHARED((N_ACC,), jnp.float32),
  ] + [pltpu.SemaphoreType.DMA] * NBUF

  def body(p_hbm, src_hbm, dst_hbm, zrow_hbm, zcnt_hbm, ones_hbm, *refs):
    if with_cnt:
      out_hbm, cnt_hbm = refs[0], refs[1]
      refs = refs[2:]
    else:
      out_hbm = refs[0]
      cnt_hbm = None
      refs = refs[1:]
    idx_s, idx_d, rows, ones_v, cbuf, s_sh, c_sh = refs[:7]
    gsem = refs[7:]

    cid = lax.axis_index("c")
    sid = lax.axis_index("s")
    row0 = sid * RPT
    base = (sid * 2 * NCHUNKS + cid * NCH0) * CHUNK
    nch = jnp.where(cid == 0, NCH0, NCH1)

    pltpu.sync_copy(zrow_hbm.at[pl.ds(row0, RPT)], s_sh.at[pl.ds(row0, RPT)])
    if with_cnt:
      pltpu.sync_copy(zcnt_hbm.at[pl.ds(row0, RPT)], cbuf)
      pltpu.sync_copy(cbuf, c_sh.at[pl.ds(row0, RPT)])
      pltpu.sync_copy(ones_hbm, ones_v)
    plsc.subcore_barrier()

    def start(g, b):
      off = base + g * CHUNK
      pltpu.sync_copy(src_hbm.at[pl.ds(off, CHUNK)], idx_s.at[b])
      pltpu.sync_copy(dst_hbm.at[pl.ds(off, CHUNK)], idx_d.at[b])
      pltpu.async_copy(p_hbm.at[idx_s.at[b]], rows.at[b], gsem[b])

    for b in range(NBUF):
      start(b, b)

    @pl.loop(0, nch, step=NBUF)
    def _steps(g0):
      for b in range(NBUF):
        g = g0 + b
        pltpu.make_async_copy(p_hbm.at[idx_s.at[b]], rows.at[b],
                              gsem[b]).wait()
        pltpu.sync_copy(rows.at[b], s_sh.at[idx_d.at[b]], add=True)
        if with_cnt:
          pltpu.sync_copy(ones_v, c_sh.at[idx_d.at[b]], add=True)

        @pl.when(g + NBUF < nch)
        def _():
          start(g + NBUF, b)

    plsc.subcore_barrier()
    pltpu.sync_copy(s_sh.at[pl.ds(row0, RPT)],
                    out_hbm.at[cid, pl.ds(row0, RPT)])
    if with_cnt:
      pltpu.sync_copy(c_sh.at[pl.ds(row0, RPT)], cbuf)
      pltpu.sync_copy(cbuf, cnt_hbm.at[pl.ds(cid * N_ACC + row0, RPT)])

  return pl.kernel(body, out_type=out_type, mesh=mesh, scratch_types=scratch)


_edge_pass_cnt = _make_edge_pass(True)
_edge_pass = _make_edge_pass(False)


BN = 2000


def _pre_body(x_ref, wl_ref, wr_ref, b_ref, t_ref, r_ref):
  x = x_ref[...]
  t_ref[...] = jnp.dot(x, wl_ref[...], preferred_element_type=jnp.float32)
  r_ref[...] = jnp.dot(x, wr_ref[...],
                       preferred_element_type=jnp.float32) + b_ref[...]


def _mid_body(s0_ref, s1_ref, c0_ref, c1_ref, r_ref, wl_ref, wr_ref, b_ref,
              t_ref, ro_ref):
  cnt = jnp.maximum(c0_ref[:, 0:1] + c1_ref[:, 0:1], 1.0)
  h = jnp.maximum((s0_ref[...] + s1_ref[...]) / cnt + r_ref[...], 0.0)
  t_ref[...] = jnp.dot(h, wl_ref[...], preferred_element_type=jnp.float32)
  ro_ref[...] = jnp.dot(h, wr_ref[...],
                        preferred_element_type=jnp.float32) + b_ref[...]


def _head_body(s0_ref, s1_ref, c0_ref, c1_ref, r_ref, wh1_ref, bh1_ref,
               wh2_ref, bh2_ref, out_ref):
  cnt = jnp.maximum(c0_ref[:, 0:1] + c1_ref[:, 0:1], 1.0)
  h = jnp.maximum((s0_ref[...] + s1_ref[...]) / cnt + r_ref[...], 0.0)
  h = jnp.maximum(jnp.dot(h, wh1_ref[...],
                          preferred_element_type=jnp.float32) + bh1_ref[...],
                  0.0)
  out_ref[...] = jnp.dot(h, wh2_ref[...],
                         preferred_element_type=jnp.float32) + bh2_ref[...]


def _row_spec(w):
  return pl.BlockSpec((BN, w), lambda i: (i, 0))


def _full_spec(shape):
  return pl.BlockSpec(shape, lambda i: (0,) * len(shape))


_GRID = N // BN

_pre = pl.pallas_call(
    _pre_body,
    grid=(_GRID,),
    in_specs=[_row_spec(D), _full_spec((D, D)), _full_spec((D, D)),
              _full_spec((1, D))],
    out_specs=[_row_spec(D), _row_spec(D)],
    out_shape=[jax.ShapeDtypeStruct((N, D), jnp.float32)] * 2,
)

_mid = pl.pallas_call(
    _mid_body,
    grid=(_GRID,),
    in_specs=[_row_spec(D), _row_spec(D), _row_spec(1), _row_spec(1),
              _row_spec(D), _full_spec((D, D)), _full_spec((D, D)),
              _full_spec((1, D))],
    out_specs=[_row_spec(D), _row_spec(D)],
    out_shape=[jax.ShapeDtypeStruct((N, D), jnp.float32)] * 2,
)

_head = pl.pallas_call(
    _head_body,
    grid=(_GRID,),
    in_specs=[_row_spec(D), _row_spec(D), _row_spec(1), _row_spec(1),
              _row_spec(D), _full_spec((D, D // 2)), _full_spec((1, D // 2)),
              _full_spec((D // 2, 4)), _full_spec((1, 4))],
    out_specs=_row_spec(4),
    out_shape=jax.ShapeDtypeStruct((N, 4), jnp.float32),
)


def kernel(x, edge_index, W_l0, b_l0, W_r0, W_l1, b_l1, W_r1, W_l2, b_l2,
           W_r2, Wh1, bh1, Wh2, bh2):
  src = edge_index[0].astype(jnp.int32)
  dst = edge_index[1].astype(jnp.int32)
  pad = NW * EPW - E
  src_p = jnp.concatenate([src, jnp.zeros((pad,), jnp.int32)])
  dst_p = jnp.concatenate([dst, jnp.full((pad,), N, jnp.int32)])
  zrow = jnp.zeros((N_ACC, D), jnp.float32)
  zcnt = jnp.zeros((N_ACC,), jnp.float32)
  ones = jnp.ones((CHUNK,), jnp.float32)

  t0, r0 = _pre(x, W_l0, W_r0, b_l0.reshape(1, D))
  s, c = _edge_pass_cnt(t0, src_p, dst_p, zrow, zcnt, ones)
  c = c.reshape(NC, N_ACC)
  c0, c1 = c[0, :N].reshape(N, 1), c[1, :N].reshape(N, 1)

  t1, r1 = _mid(s[0, :N], s[1, :N], c0, c1, r0, W_l1, W_r1,
                b_l1.reshape(1, D))
  (s,) = _edge_pass(t1, src_p, dst_p, zrow, zcnt, ones)
  t2, r2 = _mid(s[0, :N], s[1, :N], c0, c1, r1, W_l2, W_r2,
                b_l2.reshape(1, D))
  (s,) = _edge_pass(t2, src_p, dst_p, zrow, zcnt, ones)
  out = _head(s[0, :N], s[1, :N], c0, c1, r2, Wh1, bh1.reshape(1, D // 2),
              Wh2, bh2.reshape(1, 4))
  return out
